# baseline (speedup 1.0000x reference)
import jax
import jax.numpy as jnp
from jax.experimental import pallas as pl

P = 512
EMB = 32
H = 64
N = P * P
R = 2048


def _cell_kernel(corr_ref, mask_ref, ht_ref, ct_ref, wemb_ref, bemb_ref,
                 w_ref, b_ref, hout_ref, cout_ref):
    corr = corr_ref[...]
    ht = ht_ref[...]
    ct = ct_ref[...]
    w = wemb_ref[...]
    emb = jnp.maximum(
        corr[:, 0:1] * w[0:1, :] + corr[:, 1:2] * w[1:2, :] + bemb_ref[...],
        0.0)
    x = jnp.concatenate([emb, ht], axis=1)
    gates = jnp.dot(x, w_ref[...],
                    preferred_element_type=jnp.float32) + b_ref[...]
    i_g = jax.nn.sigmoid(gates[:, 0 * H:1 * H])
    f_g = jax.nn.sigmoid(gates[:, 1 * H:2 * H])
    g_g = jnp.tanh(gates[:, 2 * H:3 * H])
    o_g = jax.nn.sigmoid(gates[:, 3 * H:4 * H])
    c_new = f_g * ct + i_g * g_g
    h_new = o_g * jnp.tanh(c_new)
    m = mask_ref[...] > 0
    hout_ref[...] = jnp.where(m, h_new, ht)
    cout_ref[...] = jnp.where(m, c_new, ct)


def kernel(corr_index, rela_ht, rela_ct, nei_index, W_emb, b_emb, W_ih, b_ih,
           W_hh, b_hh):
    corr = corr_index.reshape(N, 2)
    ht = rela_ht.reshape(N, H)
    ct = rela_ct.reshape(N, H)
    mask = nei_index.reshape(N, 1)
    wembT = W_emb.T
    w = jnp.concatenate([W_ih.T, W_hh.T], axis=0)
    b = (b_ih + b_hh).reshape(1, 4 * H)
    bemb = b_emb.reshape(1, EMB)

    ht_out, ct_out = pl.pallas_call(
        _cell_kernel,
        grid=(N // R,),
        in_specs=[
            pl.BlockSpec((R, 2), lambda i: (i, 0)),
            pl.BlockSpec((R, 1), lambda i: (i, 0)),
            pl.BlockSpec((R, H), lambda i: (i, 0)),
            pl.BlockSpec((R, H), lambda i: (i, 0)),
            pl.BlockSpec((2, EMB), lambda i: (0, 0)),
            pl.BlockSpec((1, EMB), lambda i: (0, 0)),
            pl.BlockSpec((EMB + H, 4 * H), lambda i: (0, 0)),
            pl.BlockSpec((1, 4 * H), lambda i: (0, 0)),
        ],
        out_specs=[
            pl.BlockSpec((R, H), lambda i: (i, 0)),
            pl.BlockSpec((R, H), lambda i: (i, 0)),
        ],
        out_shape=[jax.ShapeDtypeStruct((N, H), jnp.float32)] * 2,
    )(corr, mask, ht, ct, wembT, bemb, w, b)
    return ht_out.reshape(P, P, H), ct_out.reshape(P, P, H)
